# Initial kernel scaffold; baseline (speedup 1.0000x reference)
import jax
import jax.numpy as jnp
from jax.experimental import pallas as pl


def kernel(x, cell_id, coordinates, nodal_values, connectivity):
    raise NotImplementedError("write your pallas kernel here")


# speedup vs baseline: 4.8052x; 4.8052x over previous
import functools

import jax
import jax.numpy as jnp
from jax import lax
from jax.experimental import pallas as pl
from jax.experimental.pallas import tpu as pltpu
from jax.experimental.pallas import tpu_sc as plsc

NC = 2
NS = 16
NW = NC * NS
L = 16


def _iota16():
    return lax.iota(jnp.int32, L)


def _full16(v):
    return jnp.full((L,), v, jnp.int32)


def _make_phase1(n_elem_pad, n_nodes, b1):
    epw = n_elem_pad // NW
    chunks = epw // b1
    mesh = plsc.VectorSubcoreMesh(core_axis_name="c", subcore_axis_name="s")

    @functools.partial(
        pl.kernel,
        out_type=jax.ShapeDtypeStruct((n_elem_pad, 9), jnp.float32),
        mesh=mesh,
        scratch_types=[
            pltpu.VMEM((b1, 3), jnp.int32),
            pltpu.VMEM((b1,), jnp.int32),
            pltpu.VMEM((b1,), jnp.int32),
            pltpu.VMEM((b1,), jnp.int32),
            pltpu.VMEM((b1, 2), jnp.float32),
            pltpu.VMEM((b1, 2), jnp.float32),
            pltpu.VMEM((b1, 2), jnp.float32),
            pltpu.VMEM((b1, 9), jnp.float32),
            pltpu.SemaphoreType.DMA,
        ],
        compiler_params=pltpu.CompilerParams(needs_layout_passes=False, use_tc_tiling_on_sc=False),
    )
    def phase1(conn_hbm, coords_hbm, m_hbm, conn_v, i1_v, i2_v, i3_v,
               c1_v, c2_v, c3_v, m_v, sem):
        wid = lax.axis_index("s") * NC + lax.axis_index("c")
        base_w = wid * epw

        @pl.loop(0, chunks)
        def _chunk(ci):
            base = base_w + ci * b1
            pltpu.sync_copy(conn_hbm.at[pl.ds(base, b1)], conn_v)

            @pl.loop(0, b1 // L)
            def _idx(i):
                rows = i * L + _iota16()
                n1 = plsc.load_gather(conn_v, [rows, _full16(0)])
                n2 = plsc.load_gather(conn_v, [rows, _full16(1)])
                n3 = plsc.load_gather(conn_v, [rows, _full16(2)])
                i1_v[pl.ds(i * L, L)] = jnp.maximum(n1 - 1, 0)
                i2_v[pl.ds(i * L, L)] = jnp.maximum(n2 - 1, 0)
                i3_v[pl.ds(i * L, L)] = jnp.maximum(n3 - 1, 0)

            pltpu.async_copy(coords_hbm.at[i1_v], c1_v, sem).wait()
            pltpu.async_copy(coords_hbm.at[i2_v], c2_v, sem).wait()
            pltpu.async_copy(coords_hbm.at[i3_v], c3_v, sem).wait()

            @pl.loop(0, b1 // L)
            def _mat(i):
                rows = i * L + _iota16()
                z = _full16(0)
                o = _full16(1)
                x1 = plsc.load_gather(c1_v, [rows, z])
                y1 = plsc.load_gather(c1_v, [rows, o])
                x2 = plsc.load_gather(c2_v, [rows, z])
                y2 = plsc.load_gather(c2_v, [rows, o])
                x3 = plsc.load_gather(c3_v, [rows, z])
                y3 = plsc.load_gather(c3_v, [rows, o])
                d1 = x1 * (y3 - y2) + x2 * (y1 - y3) + x3 * (y2 - y1)
                d2 = (-x1 * y2 + x1 * y3 + x2 * y1 - x2 * y3
                      - x3 * y1 + x3 * y2)
                d3 = x1 * (y2 - y3) + x2 * (y3 - y1) + x3 * (y1 - y2)
                vals = (
                    (y3 - y2) / d1,
                    (x2 - x3) / d2,
                    (x3 * y2 - x2 * y3) / d2,
                    (y1 - y3) / d2,
                    (x1 - x3) / d3,
                    (x3 * y1 - x1 * y3) / d3,
                    (y1 - y2) / d3,
                    (x1 - x2) / d2,
                    (x2 * y1 - x1 * y2) / d2,
                )
                for col, val in enumerate(vals):
                    plsc.store_scatter(m_v, [rows, _full16(col)], val)

            pltpu.sync_copy(m_v, m_hbm.at[pl.ds(base, b1)])

    return phase1


def _make_phase2(n_pts_pad, n_elem_pad, b2):
    ppw = n_pts_pad // NW
    chunks = ppw // b2
    mesh = plsc.VectorSubcoreMesh(core_axis_name="c", subcore_axis_name="s")

    @functools.partial(
        pl.kernel,
        out_type=jax.ShapeDtypeStruct((n_pts_pad, 3), jnp.float32),
        mesh=mesh,
        scratch_types=[
            pltpu.VMEM((b2,), jnp.int32),
            pltpu.VMEM((b2, 2), jnp.float32),
            pltpu.VMEM((b2, 9), jnp.float32),
            pltpu.VMEM((b2, 3), jnp.float32),
            pltpu.SemaphoreType.DMA,
        ],
        compiler_params=pltpu.CompilerParams(needs_layout_passes=False, use_tc_tiling_on_sc=False),
    )
    def phase2(cid_hbm, x_hbm, m_hbm, out_hbm, cid_v, x_v, m_v, o_v, sem):
        wid = lax.axis_index("s") * NC + lax.axis_index("c")
        base_w = wid * ppw

        @pl.loop(0, chunks)
        def _chunk(ci):
            base = base_w + ci * b2
            pltpu.sync_copy(cid_hbm.at[pl.ds(base, b2)], cid_v)
            pltpu.sync_copy(x_hbm.at[pl.ds(base, b2)], x_v)
            pltpu.async_copy(m_hbm.at[cid_v], m_v, sem).wait()

            @pl.loop(0, b2 // L)
            def _pt(i):
                rows = i * L + _iota16()
                x = plsc.load_gather(x_v, [rows, _full16(0)])
                y = plsc.load_gather(x_v, [rows, _full16(1)])
                m = [plsc.load_gather(m_v, [rows, _full16(c)])
                     for c in range(9)]
                o0 = x * m[0] + y * m[1] + m[2]
                o1 = x * m[3] + y * m[4] + m[5]
                o2 = x * m[6] + y * m[7] + m[8]
                plsc.store_scatter(o_v, [rows, _full16(0)], o0)
                plsc.store_scatter(o_v, [rows, _full16(1)], o1)
                plsc.store_scatter(o_v, [rows, _full16(2)], o2)

            pltpu.sync_copy(o_v, out_hbm.at[pl.ds(base, b2)])

    return phase2


def _pad_to(n, quantum):
    return -(-n // quantum) * quantum


def kernel(x, cell_id, coordinates, nodal_values, connectivity):
    del nodal_values
    n_pts = x.shape[0]
    n_elem = connectivity.shape[0]
    n_nodes = coordinates.shape[0]

    b1 = 896
    b2 = 1120
    n_elem_pad = _pad_to(n_elem, NW * b1)
    n_pts_pad = _pad_to(n_pts, NW * b2)

    coords2 = coordinates.reshape(n_nodes, 2)
    conn_pad = jnp.pad(connectivity, ((0, n_elem_pad - n_elem), (0, 0)))
    cid_pad = jnp.pad(cell_id, (0, n_pts_pad - n_pts))
    x_pad = jnp.pad(x, ((0, n_pts_pad - n_pts), (0, 0)))

    m_table = _make_phase1(n_elem_pad, n_nodes, b1)(conn_pad, coords2)
    out_pad = _make_phase2(n_pts_pad, n_elem_pad, b2)(cid_pad, x_pad, m_table)
    return out_pad[:n_pts]


# speedup vs baseline: 5.6606x; 1.1780x over previous
import functools

import jax
import jax.numpy as jnp
from jax import lax
from jax.experimental import pallas as pl
from jax.experimental.pallas import tpu as pltpu
from jax.experimental.pallas import tpu_sc as plsc

NC = 2
NS = 16
NW = NC * NS
L = 16

_SC_PARAMS = pltpu.CompilerParams(
    needs_layout_passes=False, use_tc_tiling_on_sc=False)


def _iota16():
    return lax.iota(jnp.int32, L)


def _full16(v):
    return jnp.full((L,), v, jnp.int32)


def _nchunks(n, b):
    return -(-n // b)


def _make_phase1(n_elem, n_nodes, b1):
    total_chunks = _nchunks(n_elem, b1)
    last_base = n_elem - b1
    mesh = plsc.VectorSubcoreMesh(core_axis_name="c", subcore_axis_name="s")

    @functools.partial(
        pl.kernel,
        out_type=jax.ShapeDtypeStruct((n_elem, 9), jnp.float32),
        mesh=mesh,
        scratch_types=[
            pltpu.VMEM((b1, 3), jnp.int32),
            pltpu.VMEM((b1,), jnp.int32),
            pltpu.VMEM((b1,), jnp.int32),
            pltpu.VMEM((b1,), jnp.int32),
            pltpu.VMEM((b1, 2), jnp.float32),
            pltpu.VMEM((b1, 2), jnp.float32),
            pltpu.VMEM((b1, 2), jnp.float32),
            pltpu.VMEM((b1, 9), jnp.float32),
            pltpu.SemaphoreType.DMA,
            pltpu.SemaphoreType.DMA,
            pltpu.SemaphoreType.DMA,
        ],
        compiler_params=_SC_PARAMS,
    )
    def phase1(conn_hbm, coords_hbm, m_hbm, conn_v, i1_v, i2_v, i3_v,
               c1_v, c2_v, c3_v, m_v, sem1, sem2, sem3):
        wid = lax.axis_index("s") * NC + lax.axis_index("c")

        @pl.loop(wid, total_chunks, step=NW)
        def _chunk(c):
            base = jnp.minimum(c * b1, last_base)
            pltpu.sync_copy(conn_hbm.at[pl.ds(base, b1)], conn_v)

            @pl.loop(0, b1 // L)
            def _idx(i):
                rows = i * L + _iota16()
                n1 = plsc.load_gather(conn_v, [rows, _full16(0)])
                n2 = plsc.load_gather(conn_v, [rows, _full16(1)])
                n3 = plsc.load_gather(conn_v, [rows, _full16(2)])
                i1_v[pl.ds(i * L, L)] = jnp.maximum(n1 - 1, 0)
                i2_v[pl.ds(i * L, L)] = jnp.maximum(n2 - 1, 0)
                i3_v[pl.ds(i * L, L)] = jnp.maximum(n3 - 1, 0)

            d1_ = pltpu.async_copy(coords_hbm.at[i1_v], c1_v, sem1)
            d2_ = pltpu.async_copy(coords_hbm.at[i2_v], c2_v, sem2)
            d3_ = pltpu.async_copy(coords_hbm.at[i3_v], c3_v, sem3)
            d1_.wait()
            d2_.wait()
            d3_.wait()

            @pl.loop(0, b1 // L)
            def _mat(i):
                rows = i * L + _iota16()
                z = _full16(0)
                o = _full16(1)
                x1 = plsc.load_gather(c1_v, [rows, z])
                y1 = plsc.load_gather(c1_v, [rows, o])
                x2 = plsc.load_gather(c2_v, [rows, z])
                y2 = plsc.load_gather(c2_v, [rows, o])
                x3 = plsc.load_gather(c3_v, [rows, z])
                y3 = plsc.load_gather(c3_v, [rows, o])
                d1 = x1 * (y3 - y2) + x2 * (y1 - y3) + x3 * (y2 - y1)
                d2 = (-x1 * y2 + x1 * y3 + x2 * y1 - x2 * y3
                      - x3 * y1 + x3 * y2)
                d3 = x1 * (y2 - y3) + x2 * (y3 - y1) + x3 * (y1 - y2)
                vals = (
                    (y3 - y2) / d1,
                    (x2 - x3) / d2,
                    (x3 * y2 - x2 * y3) / d2,
                    (y1 - y3) / d2,
                    (x1 - x3) / d3,
                    (x3 * y1 - x1 * y3) / d3,
                    (y1 - y2) / d3,
                    (x1 - x2) / d2,
                    (x2 * y1 - x1 * y2) / d2,
                )
                for col, val in enumerate(vals):
                    plsc.store_scatter(m_v, [rows, _full16(col)], val)

            pltpu.sync_copy(m_v, m_hbm.at[pl.ds(base, b1)])

    return phase1


def _make_phase2(n_pts, n_elem, b2):
    total_chunks = _nchunks(n_pts, b2)
    last_base = n_pts - b2
    mesh = plsc.VectorSubcoreMesh(core_axis_name="c", subcore_axis_name="s")

    @functools.partial(
        pl.kernel,
        out_type=jax.ShapeDtypeStruct((n_pts, 3), jnp.float32),
        mesh=mesh,
        scratch_types=[
            pltpu.VMEM((b2,), jnp.int32),
            pltpu.VMEM((b2, 2), jnp.float32),
            pltpu.VMEM((b2, 9), jnp.float32),
            pltpu.VMEM((b2, 3), jnp.float32),
            pltpu.SemaphoreType.DMA,
            pltpu.SemaphoreType.DMA,
        ],
        compiler_params=_SC_PARAMS,
    )
    def phase2(cid_hbm, x_hbm, m_hbm, out_hbm, cid_v, x_v, m_v, o_v,
               sem1, sem2):
        wid = lax.axis_index("s") * NC + lax.axis_index("c")

        @pl.loop(wid, total_chunks, step=NW)
        def _chunk(c):
            base = jnp.minimum(c * b2, last_base)
            dx = pltpu.async_copy(x_hbm.at[pl.ds(base, b2)], x_v, sem2)
            pltpu.sync_copy(cid_hbm.at[pl.ds(base, b2)], cid_v)
            pltpu.async_copy(m_hbm.at[cid_v], m_v, sem1).wait()
            dx.wait()

            @pl.loop(0, b2 // L)
            def _pt(i):
                rows = i * L + _iota16()
                x = plsc.load_gather(x_v, [rows, _full16(0)])
                y = plsc.load_gather(x_v, [rows, _full16(1)])
                m = [plsc.load_gather(m_v, [rows, _full16(col)])
                     for col in range(9)]
                o0 = x * m[0] + y * m[1] + m[2]
                o1 = x * m[3] + y * m[4] + m[5]
                o2 = x * m[6] + y * m[7] + m[8]
                plsc.store_scatter(o_v, [rows, _full16(0)], o0)
                plsc.store_scatter(o_v, [rows, _full16(1)], o1)
                plsc.store_scatter(o_v, [rows, _full16(2)], o2)

            pltpu.sync_copy(o_v, out_hbm.at[pl.ds(base, b2)])

    return phase2


def kernel(x, cell_id, coordinates, nodal_values, connectivity):
    del nodal_values
    n_pts = x.shape[0]
    n_elem = connectivity.shape[0]
    n_nodes = coordinates.shape[0]

    b1 = 896
    b2 = 1120

    coords2 = coordinates.reshape(n_nodes, 2)
    m_table = _make_phase1(n_elem, n_nodes, b1)(connectivity, coords2)
    return _make_phase2(n_pts, n_elem, b2)(cell_id, x, m_table)


# speedup vs baseline: 22.2206x; 3.9255x over previous
import functools

import jax
import jax.numpy as jnp
from jax import lax
from jax.experimental import pallas as pl
from jax.experimental.pallas import tpu as pltpu
from jax.experimental.pallas import tpu_sc as plsc

NC = 2
NS = 16
NW = NC * NS
L = 16

_SC_PARAMS = pltpu.CompilerParams(
    needs_layout_passes=False, use_tc_tiling_on_sc=False)


def _iota16():
    return lax.iota(jnp.int32, L)


def _full16(v):
    return jnp.full((L,), v, jnp.int32)


def _nchunks(n, b):
    return -(-n // b)


def _make_phase1(n_elem, n_nodes, b1):
    total_chunks = _nchunks(n_elem, b1)
    last_base = n_elem - b1
    mesh = plsc.VectorSubcoreMesh(core_axis_name="c", subcore_axis_name="s")

    @functools.partial(
        pl.kernel,
        out_type=jax.ShapeDtypeStruct((n_elem, 9), jnp.float32),
        mesh=mesh,
        scratch_types=[
            pltpu.VMEM((b1,), jnp.int32),
            pltpu.VMEM((b1,), jnp.int32),
            pltpu.VMEM((b1,), jnp.int32),
            pltpu.VMEM((b1, 2), jnp.float32),
            pltpu.VMEM((b1, 2), jnp.float32),
            pltpu.VMEM((b1, 2), jnp.float32),
            pltpu.VMEM((b1, 9), jnp.float32),
            pltpu.SemaphoreType.DMA,
            pltpu.SemaphoreType.DMA,
            pltpu.SemaphoreType.DMA,
        ],
        compiler_params=_SC_PARAMS,
    )
    def phase1(conn_hbm, coords_hbm, m_hbm, i1_v, i2_v, i3_v,
               c1_v, c2_v, c3_v, m_v, sem1, sem2, sem3):
        wid = lax.axis_index("s") * NC + lax.axis_index("c")

        @pl.loop(wid, total_chunks, step=NW)
        def _chunk(c):
            base = jnp.minimum(c * b1, last_base)
            pltpu.sync_copy(conn_hbm.at[pl.ds(base, b1)], i1_v)
            pltpu.sync_copy(conn_hbm.at[pl.ds(n_elem + base, b1)], i2_v)
            pltpu.sync_copy(conn_hbm.at[pl.ds(2 * n_elem + base, b1)], i3_v)

            @pl.loop(0, b1 // L)
            def _idx(i):
                s = pl.ds(i * L, L)
                i1_v[s] = jnp.maximum(i1_v[s] - 1, 0)
                i2_v[s] = jnp.maximum(i2_v[s] - 1, 0)
                i3_v[s] = jnp.maximum(i3_v[s] - 1, 0)

            d1_ = pltpu.async_copy(coords_hbm.at[i1_v], c1_v, sem1)
            d2_ = pltpu.async_copy(coords_hbm.at[i2_v], c2_v, sem2)
            d3_ = pltpu.async_copy(coords_hbm.at[i3_v], c3_v, sem3)
            d1_.wait()
            d2_.wait()
            d3_.wait()

            @pl.loop(0, b1 // L)
            def _mat(i):
                rows = i * L + _iota16()
                z = _full16(0)
                o = _full16(1)
                x1 = plsc.load_gather(c1_v, [rows, z])
                y1 = plsc.load_gather(c1_v, [rows, o])
                x2 = plsc.load_gather(c2_v, [rows, z])
                y2 = plsc.load_gather(c2_v, [rows, o])
                x3 = plsc.load_gather(c3_v, [rows, z])
                y3 = plsc.load_gather(c3_v, [rows, o])
                d1 = x1 * (y3 - y2) + x2 * (y1 - y3) + x3 * (y2 - y1)
                d2 = (-x1 * y2 + x1 * y3 + x2 * y1 - x2 * y3
                      - x3 * y1 + x3 * y2)
                d3 = x1 * (y2 - y3) + x2 * (y3 - y1) + x3 * (y1 - y2)
                vals = (
                    (y3 - y2) / d1,
                    (x2 - x3) / d2,
                    (x3 * y2 - x2 * y3) / d2,
                    (y1 - y3) / d2,
                    (x1 - x3) / d3,
                    (x3 * y1 - x1 * y3) / d3,
                    (y1 - y2) / d3,
                    (x1 - x2) / d2,
                    (x2 * y1 - x1 * y2) / d2,
                )
                for col, val in enumerate(vals):
                    plsc.store_scatter(m_v, [rows, _full16(col)], val)

            pltpu.sync_copy(m_v, m_hbm.at[pl.ds(base, b1)])

    return phase1


def _make_phase2(n_pts, n_elem, b2):
    total_chunks = _nchunks(n_pts, b2)
    last_base = n_pts - b2
    mesh = plsc.VectorSubcoreMesh(core_axis_name="c", subcore_axis_name="s")
    out_sds = jax.ShapeDtypeStruct((n_pts,), jnp.float32)

    @functools.partial(
        pl.kernel,
        out_type=(out_sds, out_sds, out_sds),
        mesh=mesh,
        scratch_types=[
            pltpu.VMEM((b2,), jnp.int32),
            pltpu.VMEM((b2,), jnp.float32),
            pltpu.VMEM((b2,), jnp.float32),
            pltpu.VMEM((b2, 9), jnp.float32),
            pltpu.VMEM((b2,), jnp.float32),
            pltpu.VMEM((b2,), jnp.float32),
            pltpu.VMEM((b2,), jnp.float32),
            pltpu.SemaphoreType.DMA,
            pltpu.SemaphoreType.DMA,
            pltpu.SemaphoreType.DMA,
        ],
        compiler_params=_SC_PARAMS,
    )
    def phase2(cid_hbm, xt_hbm, m_hbm, o0_hbm, o1_hbm, o2_hbm,
               cid_v, x_v, y_v, m_v, o0_v, o1_v, o2_v, sem1, sem2, sem3):
        wid = lax.axis_index("s") * NC + lax.axis_index("c")

        @pl.loop(wid, total_chunks, step=NW)
        def _chunk(c):
            base = jnp.minimum(c * b2, last_base)
            dx = pltpu.async_copy(xt_hbm.at[pl.ds(base, b2)], x_v, sem2)
            dy = pltpu.async_copy(xt_hbm.at[pl.ds(n_pts + base, b2)], y_v,
                                  sem3)
            pltpu.sync_copy(cid_hbm.at[pl.ds(base, b2)], cid_v)
            pltpu.async_copy(m_hbm.at[cid_v], m_v, sem1).wait()
            dx.wait()
            dy.wait()

            @pl.loop(0, b2 // L)
            def _pt(i):
                s = pl.ds(i * L, L)
                rows = i * L + _iota16()
                x = x_v[s]
                y = y_v[s]
                m = [plsc.load_gather(m_v, [rows, _full16(col)])
                     for col in range(9)]
                o0_v[s] = x * m[0] + y * m[1] + m[2]
                o1_v[s] = x * m[3] + y * m[4] + m[5]
                o2_v[s] = x * m[6] + y * m[7] + m[8]

            pltpu.sync_copy(o0_v, o0_hbm.at[pl.ds(base, b2)])
            pltpu.sync_copy(o1_v, o1_hbm.at[pl.ds(base, b2)])
            pltpu.sync_copy(o2_v, o2_hbm.at[pl.ds(base, b2)])

    return phase2


def kernel(x, cell_id, coordinates, nodal_values, connectivity):
    del nodal_values
    n_pts = x.shape[0]
    n_elem = connectivity.shape[0]
    n_nodes = coordinates.shape[0]

    b1 = 896
    b2 = 1120

    coords2 = coordinates.reshape(n_nodes, 2)
    conn_t = connectivity.T.reshape(3 * n_elem)
    xt = x.T.reshape(2 * n_pts)

    m_table = _make_phase1(n_elem, n_nodes, b1)(conn_t, coords2)
    o0, o1, o2 = _make_phase2(n_pts, n_elem, b2)(cell_id, xt, m_table)
    return jnp.stack([o0, o1, o2], axis=1)
